# baseline (speedup 1.0000x reference)
import jax
import jax.numpy as jnp
from jax.experimental import pallas as pl


def _body(x_ref, slow_ref, fast_ref):
    v = x_ref[...]
    fast_ref[...] = v
    slow_ref[...] = v


def kernel(frames):
    C, T, H, W = frames.shape
    S = T // 4
    slow, fast = pl.pallas_call(
        _body,
        grid=(C, T),
        in_specs=[pl.BlockSpec((1, 1, H, W), lambda c, t: (c, t, 0, 0))],
        out_specs=[
            pl.BlockSpec(
                (1, 1, H, W),
                lambda c, t: (c, (t * (S - 1) + (T - 2)) // (T - 1), 0, 0),
            ),
            pl.BlockSpec((1, 1, H, W), lambda c, t: (c, t, 0, 0)),
        ],
        out_shape=[
            jax.ShapeDtypeStruct((C, S, H, W), frames.dtype),
            jax.ShapeDtypeStruct((C, T, H, W), frames.dtype),
        ],
    )(frames)
    return (slow, fast)
